# baseline (speedup 1.0000x reference)
import functools

import jax
import jax.numpy as jnp
from jax import lax
from jax.experimental import pallas as pl
from jax.experimental.pallas import tpu as pltpu
from jax.experimental.pallas import tpu_sc as plsc

B = 4096
L = 50
D = 64
NC = 2
NS = 16
NW = NC * NS
BPW = B // NW
CH = 8
NCHUNK = BPW // CH
VL = 32
DK = D // VL


def _sc_gather_sum(x0, x1, x2, w0, w1, w2):
    mesh = plsc.VectorSubcoreMesh(core_axis_name="c", subcore_axis_name="s")

    @functools.partial(
        pl.kernel,
        out_type=jax.ShapeDtypeStruct((B, D), jnp.bfloat16),
        mesh=mesh,
        scratch_types=[
            pltpu.VMEM((BPW * L,), jnp.int32),
            pltpu.VMEM((BPW * L,), jnp.int32),
            pltpu.VMEM((BPW * L,), jnp.int32),
            pltpu.VMEM((CH * L, D), jnp.bfloat16),
            pltpu.VMEM((CH * L, D), jnp.bfloat16),
            pltpu.VMEM((BPW, D), jnp.bfloat16),
            pltpu.SemaphoreType.DMA,
            pltpu.SemaphoreType.DMA,
        ],
        compiler_params=pltpu.CompilerParams(use_tc_tiling_on_sc=False),
    )
    def k(x0h, x1h, x2h, w0h, w1h, w2h, out_h,
          idx0, idx1, idx2, rowsA, rowsB, acc, semA, semB):
        wid = lax.axis_index("s") * NC + lax.axis_index("c")
        base = wid * BPW

        pltpu.sync_copy(x0h.at[pl.ds(base * L, BPW * L)], idx0)
        pltpu.sync_copy(x1h.at[pl.ds(base * L, BPW * L)], idx1)
        pltpu.sync_copy(x2h.at[pl.ds(base * L, BPW * L)], idx2)

        for t, (tbl, idxv) in enumerate(((w0h, idx0), (w1h, idx1), (w2h, idx2))):

            def issue(j, buf, sem):
                pltpu.async_copy(
                    tbl.at[idxv.at[pl.ds(j * CH * L, CH * L)]], buf, sem)

            def wait(buf, sem):
                pltpu.make_async_copy(
                    tbl.at[idxv.at[pl.ds(0, CH * L)]], buf, sem).wait()

            def accum(j, buf):
                for c in range(CH):
                    row = j * CH + c
                    init = tuple(buf[c * L, pl.ds(VL * kk, VL)]
                                 for kk in range(DK))

                    def body(l, carry):
                        return tuple(
                            carry[kk] + buf[c * L + l, pl.ds(VL * kk, VL)]
                            for kk in range(DK))

                    sums = lax.fori_loop(1, L, body, init)
                    for kk in range(DK):
                        sl = (row, pl.ds(VL * kk, VL))
                        if t == 0:
                            acc[sl] = sums[kk]
                        else:
                            acc[sl] = acc[sl] + sums[kk]

            issue(0, rowsA, semA)

            @pl.loop(0, NCHUNK - 2, step=2)
            def _(jj):
                issue(jj + 1, rowsB, semB)
                wait(rowsA, semA)
                accum(jj, rowsA)
                issue(jj + 2, rowsA, semA)
                wait(rowsB, semB)
                accum(jj + 1, rowsB)

            issue(NCHUNK - 1, rowsB, semB)
            wait(rowsA, semA)
            accum(NCHUNK - 2, rowsA)
            wait(rowsB, semB)
            accum(NCHUNK - 1, rowsB)

        pltpu.sync_copy(acc, out_h.at[pl.ds(base, BPW), :])

    return k(x0, x1, x2, w0, w1, w2)


def _tc_mlp_softmax(e_sum, w_h, b_h, w_o, b_o):
    def body(e_ref, wh_ref, bh_ref, wo_ref, bo_ref, o_ref):
        wo = wo_ref[...]
        wc = jnp.dot(wh_ref[...], wo, preferred_element_type=jnp.float32)
        bc = jnp.dot(bh_ref[...], wo, preferred_element_type=jnp.float32) \
            + bo_ref[...]
        e_avg = e_ref[...].astype(jnp.float32) * (1.0 / (3 * L))
        logits = jnp.dot(e_avg, wc, preferred_element_type=jnp.float32) + bc
        m = jnp.max(logits, axis=1, keepdims=True)
        ex = jnp.exp(logits - m)
        o_ref[...] = ex / jnp.sum(ex, axis=1, keepdims=True)

    return pl.pallas_call(
        body,
        out_shape=jax.ShapeDtypeStruct((B, b_o.shape[-1]), jnp.float32),
    )(e_sum, w_h, b_h, w_o, b_o)


def kernel(x_0, x_1, x_2, W_word, W_2gram, W_3gram, W_h, b_h, W_o, b_o):
    x_0 = x_0.astype(jnp.int32).reshape(-1)
    x_1 = x_1.astype(jnp.int32).reshape(-1)
    x_2 = x_2.astype(jnp.int32).reshape(-1)
    e_sum = _sc_gather_sum(x_0, x_1, x_2,
                           W_word.astype(jnp.bfloat16),
                           W_2gram.astype(jnp.bfloat16),
                           W_3gram.astype(jnp.bfloat16))
    return _tc_mlp_softmax(e_sum, W_h.astype(jnp.float32),
                           b_h.reshape(1, -1).astype(jnp.float32),
                           W_o.astype(jnp.float32),
                           b_o.reshape(1, -1).astype(jnp.float32))


# speedup vs baseline: 1.2273x; 1.2273x over previous
import functools

import jax
import jax.numpy as jnp
from jax import lax
from jax.experimental import pallas as pl
from jax.experimental.pallas import tpu as pltpu
from jax.experimental.pallas import tpu_sc as plsc

B = 4096
L = 50
D = 64
NC = 2
NS = 16
NW = NC * NS
BPW = B // NW
CH = 8
NCHUNK = BPW // CH
VL = 16
DK = D // VL


def _sc_gather_sum(x0, x1, x2, w0, w1, w2):
    mesh = plsc.VectorSubcoreMesh(core_axis_name="c", subcore_axis_name="s")

    @functools.partial(
        pl.kernel,
        out_type=jax.ShapeDtypeStruct((B, D), jnp.float32),
        mesh=mesh,
        scratch_types=[
            pltpu.VMEM((BPW * L,), jnp.int32),
            pltpu.VMEM((BPW * L,), jnp.int32),
            pltpu.VMEM((BPW * L,), jnp.int32),
            pltpu.VMEM((CH * L, D), jnp.float32),
            pltpu.VMEM((CH * L, D), jnp.float32),
            pltpu.VMEM((BPW, D), jnp.float32),
            pltpu.SemaphoreType.DMA,
            pltpu.SemaphoreType.DMA,
        ],
        compiler_params=pltpu.CompilerParams(use_tc_tiling_on_sc=False),
    )
    def k(x0h, x1h, x2h, w0h, w1h, w2h, out_h,
          idx0, idx1, idx2, rowsA, rowsB, acc, semA, semB):
        wid = lax.axis_index("s") * NC + lax.axis_index("c")
        base = wid * BPW

        pltpu.sync_copy(x0h.at[pl.ds(base * L, BPW * L)], idx0)
        pltpu.sync_copy(x1h.at[pl.ds(base * L, BPW * L)], idx1)
        pltpu.sync_copy(x2h.at[pl.ds(base * L, BPW * L)], idx2)

        for t, (tbl, idxv) in enumerate(((w0h, idx0), (w1h, idx1), (w2h, idx2))):

            def issue(j, buf, sem):
                pltpu.async_copy(
                    tbl.at[idxv.at[pl.ds(j * CH * L, CH * L)]], buf, sem)

            def wait(buf, sem):
                pltpu.make_async_copy(
                    tbl.at[idxv.at[pl.ds(0, CH * L)]], buf, sem).wait()

            def accum(j, buf):
                for c in range(CH):
                    row = j * CH + c
                    init = tuple(buf[c * L, pl.ds(VL * kk, VL)]
                                 for kk in range(DK))

                    def body(l, carry):
                        return tuple(
                            carry[kk] + buf[c * L + l, pl.ds(VL * kk, VL)]
                            for kk in range(DK))

                    sums = lax.fori_loop(1, L, body, init)
                    for kk in range(DK):
                        sl = (row, pl.ds(VL * kk, VL))
                        if t == 0:
                            acc[sl] = sums[kk]
                        else:
                            acc[sl] = acc[sl] + sums[kk]

            issue(0, rowsA, semA)

            @pl.loop(0, NCHUNK - 2, step=2)
            def _(jj):
                issue(jj + 1, rowsB, semB)
                wait(rowsA, semA)
                accum(jj, rowsA)
                issue(jj + 2, rowsA, semA)
                wait(rowsB, semB)
                accum(jj + 1, rowsB)

            issue(NCHUNK - 1, rowsB, semB)
            wait(rowsA, semA)
            accum(NCHUNK - 2, rowsA)
            wait(rowsB, semB)
            accum(NCHUNK - 1, rowsB)

        pltpu.sync_copy(acc, out_h.at[pl.ds(base, BPW), :])

    return k(x0, x1, x2, w0, w1, w2)


def _tc_mlp_softmax(e_sum, w_h, b_h, w_o, b_o):
    def body(e_ref, wh_ref, bh_ref, wo_ref, bo_ref, o_ref):
        wo = wo_ref[...]
        wc = jnp.dot(wh_ref[...], wo, preferred_element_type=jnp.float32)
        bc = jnp.dot(bh_ref[...], wo, preferred_element_type=jnp.float32) \
            + bo_ref[...]
        e_avg = e_ref[...] * (1.0 / (3 * L))
        logits = jnp.dot(e_avg, wc, preferred_element_type=jnp.float32) + bc
        m = jnp.max(logits, axis=1, keepdims=True)
        ex = jnp.exp(logits - m)
        o_ref[...] = ex / jnp.sum(ex, axis=1, keepdims=True)

    return pl.pallas_call(
        body,
        out_shape=jax.ShapeDtypeStruct((B, b_o.shape[-1]), jnp.float32),
    )(e_sum, w_h, b_h, w_o, b_o)


def kernel(x_0, x_1, x_2, W_word, W_2gram, W_3gram, W_h, b_h, W_o, b_o):
    x_0 = x_0.astype(jnp.int32).reshape(-1)
    x_1 = x_1.astype(jnp.int32).reshape(-1)
    x_2 = x_2.astype(jnp.int32).reshape(-1)
    W_word, W_2gram, W_3gram = [
        jax.lax.optimization_barrier(w.reshape(-1)).reshape(w.shape)
        for w in (W_word, W_2gram, W_3gram)]
    e_sum = _sc_gather_sum(x_0, x_1, x_2, W_word, W_2gram, W_3gram)
    return _tc_mlp_softmax(e_sum, W_h.astype(jnp.float32),
                           b_h.reshape(1, -1).astype(jnp.float32),
                           W_o.astype(jnp.float32),
                           b_o.reshape(1, -1).astype(jnp.float32))
